# baseline (speedup 1.0000x reference)
import functools

import jax
import jax.numpy as jnp
from jax import lax
from jax.experimental import pallas as pl
from jax.experimental.pallas import tpu as pltpu
from jax.experimental.pallas import tpu_sc as plsc

EMBED = 128
N_TOK = 1024 * 200
FANIN = 5
NW = 32
TOK_PER_W = N_TOK // NW
PAIR = 128
N_PAIR = TOK_PER_W // PAIR
CHUNK = 64
ROWS_PER_CHUNK = FANIN * CHUNK
TABLE_ROWS = 1001 + 7 + 3 * 128
IDX_STAGE = 128
PAIR_PER_STAGE = N_PAIR // 2

_mesh = plsc.VectorSubcoreMesh(core_axis_name="c", subcore_axis_name="s")


@functools.partial(
    pl.kernel,
    mesh=_mesh,
    out_type=jax.ShapeDtypeStruct((N_TOK, EMBED), jnp.float32),
    scratch_types=[
        pltpu.VMEM((IDX_STAGE, EMBED), jnp.int32),
        pltpu.VMEM((ROWS_PER_CHUNK, EMBED), jnp.float32),
        pltpu.VMEM((ROWS_PER_CHUNK, EMBED), jnp.float32),
        pltpu.VMEM((2, CHUNK), jnp.int32),
        pltpu.VMEM_SHARED((TABLE_ROWS, EMBED), jnp.float32),
        pltpu.VMEM_SHARED((NW * CHUNK, EMBED), jnp.float32),
        pltpu.SemaphoreType.DMA,
        pltpu.SemaphoreType.DMA,
        pltpu.SemaphoreType.DMA,
        pltpu.SemaphoreType.DMA,
        pltpu.SemaphoreType.DMA,
    ],
)
def _emb_kernel(table_hbm, idx_hbm, out_hbm, idx_v, buf_a, buf_b,
                id64_v, table_sh, acc_sh,
                sem_ga, sem_gb, sem_s, sem_oa, sem_ob):
    sid = lax.axis_index("s")
    wid = sid * 2 + lax.axis_index("c")
    tok_base = wid * TOK_PER_W

    @pl.when(sid == 0)
    def _():
        pltpu.sync_copy(table_hbm, table_sh)

    for x in range(2):
        base = (sid * 2 + x) * CHUNK
        for v in range(CHUNK // 16):
            iv = lax.iota(jnp.int32, 16) + (v * 16 + base)
            id64_v[x, pl.ds(v * 16, 16)] = iv
    plsc.subcore_barrier()

    def issue_gathers(p_local, half, buf, sem):
        for j in range(FANIN):
            pltpu.async_copy(
                table_sh.at[idx_v.at[p_local * FANIN + j,
                                     pl.ds(half * CHUNK, CHUNK)]],
                buf.at[pl.ds(j * CHUNK, CHUNK)],
                sem,
            )

    def wait_gathers(p_local, half, buf, sem):
        for j in range(FANIN):
            pltpu.make_async_copy(
                table_sh.at[idx_v.at[p_local * FANIN + j,
                                     pl.ds(half * CHUNK, CHUNK)]],
                buf.at[pl.ds(j * CHUNK, CHUNK)],
                sem,
            ).wait()

    def reduce_chunk(buf, x):
        pltpu.sync_copy(buf.at[pl.ds(0, CHUNK)], acc_sh.at[id64_v.at[x]])
        for j in range(1, FANIN):
            pltpu.async_copy(
                buf.at[pl.ds(j * CHUNK, CHUNK)],
                acc_sh.at[id64_v.at[x]], sem_s, add=True).wait()

    def issue_out(chunk_idx, x, sem):
        pltpu.async_copy(
            acc_sh.at[pl.ds((sid * 2 + x) * CHUNK, CHUNK)],
            out_hbm.at[pl.ds(tok_base + chunk_idx * CHUNK, CHUNK)],
            sem,
        )

    def wait_out(chunk_idx, x, sem):
        pltpu.make_async_copy(
            acc_sh.at[pl.ds((sid * 2 + x) * CHUNK, CHUNK)],
            out_hbm.at[pl.ds(tok_base + chunk_idx * CHUNK, CHUNK)],
            sem,
        ).wait()

    for stage in range(2):
        pair0 = stage * PAIR_PER_STAGE
        pltpu.sync_copy(idx_hbm.at[wid, stage], idx_v)
        issue_gathers(0, 0, buf_a, sem_ga)

        def pair_body(pp, carry):
            p = pair0 + pp
            even = 2 * p
            issue_gathers(pp, 1, buf_b, sem_gb)

            wait_gathers(pp, 0, buf_a, sem_ga)
            reduce_chunk(buf_a, 0)
            issue_out(even, 0, sem_oa)
            wait_out(even, 0, sem_oa)

            @pl.when(pp < PAIR_PER_STAGE - 1)
            def _():
                issue_gathers(pp + 1, 0, buf_a, sem_ga)

            wait_gathers(pp, 1, buf_b, sem_gb)
            reduce_chunk(buf_b, 1)
            issue_out(even + 1, 1, sem_ob)
            wait_out(even + 1, 1, sem_ob)
            return carry

        lax.fori_loop(0, PAIR_PER_STAGE, pair_body, 0)


def kernel(value, depth, position, W_value, W_depth, W_pos):
    nv = W_value.shape[0]
    nd = W_depth.shape[0]
    npos = W_pos.shape[1]
    table = jnp.concatenate(
        [W_value, W_depth, W_pos[0], W_pos[1], W_pos[2]], axis=0)
    off_d = nv
    off_p = nv + nd
    idx = jnp.stack(
        [
            value.reshape(-1),
            depth.reshape(-1) + off_d,
            position[..., 0].reshape(-1) + off_p,
            position[..., 1].reshape(-1) + (off_p + npos),
            position[..., 2].reshape(-1) + (off_p + 2 * npos),
        ],
        axis=1,
    ).reshape(NW, N_PAIR, PAIR, FANIN).transpose(0, 1, 3, 2)
    idx = idx.reshape(NW, N_PAIR * FANIN, EMBED).astype(jnp.int32)
    idx = jnp.pad(idx.reshape(NW, 2, N_PAIR * FANIN // 2, EMBED),
                  ((0, 0), (0, 0), (0, IDX_STAGE - N_PAIR * FANIN // 2), (0, 0)))
    out = _emb_kernel(table, idx)
    return out.reshape(value.shape[0], value.shape[1], EMBED)


# speedup vs baseline: 1.3082x; 1.3082x over previous
import functools

import jax
import jax.numpy as jnp
from jax import lax
from jax.experimental import pallas as pl
from jax.experimental.pallas import tpu as pltpu
from jax.experimental.pallas import tpu_sc as plsc

EMBED = 128
N_TOK = 1024 * 200
FANIN = 5
NW = 32
TOK_PER_W = N_TOK // NW
PAIR = 128
N_PAIR = TOK_PER_W // PAIR
CHUNK = 64
ROWS_PER_CHUNK = FANIN * CHUNK
TABLE_ROWS = 1001 + 7 + 3 * 128
IDX_STAGE = 128
PAIR_PER_STAGE = N_PAIR // 2

_mesh = plsc.VectorSubcoreMesh(core_axis_name="c", subcore_axis_name="s")


@functools.partial(
    pl.kernel,
    mesh=_mesh,
    out_type=jax.ShapeDtypeStruct((N_TOK, EMBED), jnp.float32),
    scratch_types=[
        pltpu.VMEM((IDX_STAGE, EMBED), jnp.int32),
        pltpu.VMEM((ROWS_PER_CHUNK, EMBED), jnp.float32),
        pltpu.VMEM((ROWS_PER_CHUNK, EMBED), jnp.float32),
        pltpu.VMEM((2, CHUNK), jnp.int32),
        pltpu.VMEM_SHARED((TABLE_ROWS, EMBED), jnp.float32),
        pltpu.VMEM_SHARED((NW * CHUNK, EMBED), jnp.float32),
        pltpu.SemaphoreType.DMA,
        pltpu.SemaphoreType.DMA,
        pltpu.SemaphoreType.DMA,
        pltpu.SemaphoreType.DMA,
        pltpu.SemaphoreType.DMA,
        pltpu.SemaphoreType.DMA,
    ],
)
def _emb_kernel(table_hbm, idx_hbm, out_hbm, idx_v, buf_a, buf_b,
                id64_v, table_sh, acc_sh,
                sem_ga, sem_gb, sem_sa, sem_sb, sem_oa, sem_ob):
    sid = lax.axis_index("s")
    wid = sid * 2 + lax.axis_index("c")
    tok_base = wid * TOK_PER_W

    @pl.when(sid == 0)
    def _():
        pltpu.sync_copy(table_hbm, table_sh)

    for x in range(2):
        base = (sid * 2 + x) * CHUNK
        for v in range(CHUNK // 16):
            iv = lax.iota(jnp.int32, 16) + (v * 16 + base)
            id64_v[x, pl.ds(v * 16, 16)] = iv
    plsc.subcore_barrier()

    def issue_gathers(p_local, half, buf, sem):
        for j in range(FANIN):
            pltpu.async_copy(
                table_sh.at[idx_v.at[p_local * FANIN + j,
                                     pl.ds(half * CHUNK, CHUNK)]],
                buf.at[pl.ds(j * CHUNK, CHUNK)],
                sem,
            )

    def wait_gathers(p_local, half, buf, sem):
        for j in range(FANIN):
            pltpu.make_async_copy(
                table_sh.at[idx_v.at[p_local * FANIN + j,
                                     pl.ds(half * CHUNK, CHUNK)]],
                buf.at[pl.ds(j * CHUNK, CHUNK)],
                sem,
            ).wait()

    def scatter(buf, j, x, sem, add):
        return pltpu.async_copy(
            buf.at[pl.ds(j * CHUNK, CHUNK)],
            acc_sh.at[id64_v.at[x]], sem, add=add)

    def wait_scatter(buf, j, x, sem, add):
        del add
        pltpu.make_async_copy(
            buf.at[pl.ds(j * CHUNK, CHUNK)],
            acc_sh.at[id64_v.at[x]], sem).wait()

    def compute(buf, lo, hi):
        def tok_body(t, carry):
            for v in range(EMBED // 16):
                sl = pl.ds(v * 16, 16)
                acc = buf[t, sl] + buf[CHUNK + t, sl]
                acc = acc + buf[2 * CHUNK + t, sl]
                buf[t, sl] = acc
            return carry
        lax.fori_loop(lo, hi, tok_body, 0)

    def issue_out(chunk_idx, x, sem):
        pltpu.async_copy(
            acc_sh.at[pl.ds((sid * 2 + x) * CHUNK, CHUNK)],
            out_hbm.at[pl.ds(tok_base + chunk_idx * CHUNK, CHUNK)],
            sem,
        )

    def wait_out(chunk_idx, x, sem):
        pltpu.make_async_copy(
            acc_sh.at[pl.ds((sid * 2 + x) * CHUNK, CHUNK)],
            out_hbm.at[pl.ds(tok_base + chunk_idx * CHUNK, CHUNK)],
            sem,
        ).wait()

    for stage in range(2):
        pair0 = stage * PAIR_PER_STAGE
        pltpu.sync_copy(idx_hbm.at[wid, stage], idx_v)
        issue_gathers(0, 0, buf_a, sem_ga)

        def pair_body(pp, carry):
            p = pair0 + pp
            even = 2 * p

            if stage == 0:
                @pl.when(pp > 0)
                def _():
                    wait_scatter(buf_b, 0, 1, sem_sb, True)
                    issue_out(even - 1, 1, sem_ob)
            else:
                wait_scatter(buf_b, 0, 1, sem_sb, True)
                issue_out(even - 1, 1, sem_ob)
            issue_gathers(pp, 1, buf_b, sem_gb)

            wait_gathers(pp, 0, buf_a, sem_ga)
            if stage == 0:
                @pl.when(pp > 0)
                def _():
                    wait_out(even - 2, 0, sem_oa)
            else:
                wait_out(even - 2, 0, sem_oa)
            scatter(buf_a, 3, 0, sem_sa, False)
            compute(buf_a, 0, CHUNK // 2)
            wait_scatter(buf_a, 3, 0, sem_sa, False)
            scatter(buf_a, 4, 0, sem_sa, True)
            compute(buf_a, CHUNK // 2, CHUNK)
            wait_scatter(buf_a, 4, 0, sem_sa, True)
            scatter(buf_a, 0, 0, sem_sa, True)

            wait_gathers(pp, 1, buf_b, sem_gb)
            if stage == 0:
                @pl.when(pp > 0)
                def _():
                    wait_out(even - 1, 1, sem_ob)
            else:
                wait_out(even - 1, 1, sem_ob)
            scatter(buf_b, 3, 1, sem_sb, False)
            compute(buf_b, 0, CHUNK // 2)
            wait_scatter(buf_b, 3, 1, sem_sb, False)
            scatter(buf_b, 4, 1, sem_sb, True)
            compute(buf_b, CHUNK // 2, CHUNK)

            wait_scatter(buf_a, 0, 0, sem_sa, True)
            issue_out(even, 0, sem_oa)
            @pl.when(pp < PAIR_PER_STAGE - 1)
            def _():
                issue_gathers(pp + 1, 0, buf_a, sem_ga)

            wait_scatter(buf_b, 4, 1, sem_sb, True)
            scatter(buf_b, 0, 1, sem_sb, True)
            return carry

        lax.fori_loop(0, PAIR_PER_STAGE, pair_body, 0)

    wait_scatter(buf_b, 0, 1, sem_sb, True)
    issue_out(2 * N_PAIR - 1, 1, sem_ob)
    wait_out(2 * N_PAIR - 2, 0, sem_oa)
    wait_out(2 * N_PAIR - 1, 1, sem_ob)


def kernel(value, depth, position, W_value, W_depth, W_pos):
    nv = W_value.shape[0]
    nd = W_depth.shape[0]
    npos = W_pos.shape[1]
    table = jnp.concatenate(
        [W_value, W_depth, W_pos[0], W_pos[1], W_pos[2]], axis=0)
    off_d = nv
    off_p = nv + nd
    idx = jnp.stack(
        [
            value.reshape(-1),
            depth.reshape(-1) + off_d,
            position[..., 0].reshape(-1) + off_p,
            position[..., 1].reshape(-1) + (off_p + npos),
            position[..., 2].reshape(-1) + (off_p + 2 * npos),
        ],
        axis=1,
    ).reshape(NW, N_PAIR, PAIR, FANIN).transpose(0, 1, 3, 2)
    idx = idx.reshape(NW, N_PAIR * FANIN, EMBED).astype(jnp.int32)
    idx = jnp.pad(idx.reshape(NW, 2, N_PAIR * FANIN // 2, EMBED),
                  ((0, 0), (0, 0), (0, IDX_STAGE - N_PAIR * FANIN // 2), (0, 0)))
    out = _emb_kernel(table, idx)
    return out.reshape(value.shape[0], value.shape[1], EMBED)


# speedup vs baseline: 1.5432x; 1.1796x over previous
import functools

import jax
import jax.numpy as jnp
from jax import lax
from jax.experimental import pallas as pl
from jax.experimental.pallas import tpu as pltpu
from jax.experimental.pallas import tpu_sc as plsc

EMBED = 128
N_TOK = 1024 * 200
FANIN = 4
NW = 32
TOK_PER_W = N_TOK // NW
PAIR = 128
N_PAIR = TOK_PER_W // PAIR
CHUNK = 64
ROWS_PER_CHUNK = FANIN * CHUNK
N_VD = 1001 * 7
TABLE_ROWS = N_VD + 3 * 128
N_STAGE = 5
PAIR_PER_STAGE = N_PAIR // N_STAGE
IDX_STAGE = PAIR_PER_STAGE * FANIN

_mesh = plsc.VectorSubcoreMesh(core_axis_name="c", subcore_axis_name="s")


@functools.partial(
    pl.kernel,
    mesh=_mesh,
    out_type=jax.ShapeDtypeStruct((N_TOK, EMBED), jnp.float32),
    scratch_types=[
        pltpu.VMEM((IDX_STAGE, EMBED), jnp.int32),
        pltpu.VMEM((ROWS_PER_CHUNK, EMBED), jnp.float32),
        pltpu.VMEM((ROWS_PER_CHUNK, EMBED), jnp.float32),
        pltpu.VMEM_SHARED((TABLE_ROWS, EMBED), jnp.float32),
        pltpu.SemaphoreType.DMA,
        pltpu.SemaphoreType.DMA,
        pltpu.SemaphoreType.DMA,
        pltpu.SemaphoreType.DMA,
    ],
)
def _emb_kernel(table_hbm, idx_hbm, out_hbm, idx_v, buf_a, buf_b,
                table_sh, sem_a, sem_b, sem_oa, sem_ob):
    sid = lax.axis_index("s")
    wid = sid * 2 + lax.axis_index("c")
    tok_base = wid * TOK_PER_W

    @pl.when(sid == 0)
    def _():
        pltpu.sync_copy(table_hbm, table_sh)
    plsc.subcore_barrier()

    def issue_gathers(p_local, half, buf, sem):
        for j in range(FANIN):
            pltpu.async_copy(
                table_sh.at[idx_v.at[p_local * FANIN + j,
                                     pl.ds(half * CHUNK, CHUNK)]],
                buf.at[pl.ds(j * CHUNK, CHUNK)],
                sem,
            )

    def wait_gathers(p_local, half, buf, sem):
        for j in range(FANIN):
            pltpu.make_async_copy(
                table_sh.at[idx_v.at[p_local * FANIN + j,
                                     pl.ds(half * CHUNK, CHUNK)]],
                buf.at[pl.ds(j * CHUNK, CHUNK)],
                sem,
            ).wait()

    def compute(buf):
        def tok_body(t, carry):
            for v in range(EMBED // 16):
                sl = pl.ds(v * 16, 16)
                acc = (buf[t, sl] + buf[CHUNK + t, sl]) + (
                    buf[2 * CHUNK + t, sl] + buf[3 * CHUNK + t, sl])
                buf[t, sl] = acc
            return carry
        lax.fori_loop(0, CHUNK, tok_body, 0)

    def issue_out(chunk_idx, buf, sem):
        pltpu.async_copy(
            buf.at[pl.ds(0, CHUNK)],
            out_hbm.at[pl.ds(tok_base + chunk_idx * CHUNK, CHUNK)],
            sem,
        )

    def wait_out(chunk_idx, buf, sem):
        pltpu.make_async_copy(
            buf.at[pl.ds(0, CHUNK)],
            out_hbm.at[pl.ds(tok_base + chunk_idx * CHUNK, CHUNK)],
            sem,
        ).wait()

    for stage in range(N_STAGE):
        pair0 = stage * PAIR_PER_STAGE
        pltpu.sync_copy(idx_hbm.at[wid, stage], idx_v)
        issue_gathers(0, 0, buf_a, sem_a)

        def pair_body(pp, carry):
            p = pair0 + pp
            even = 2 * p
            if stage == 0:
                @pl.when(pp > 0)
                def _():
                    wait_out(even - 1, buf_b, sem_ob)
            else:
                wait_out(even - 1, buf_b, sem_ob)
            issue_gathers(pp, 1, buf_b, sem_b)

            wait_gathers(pp, 0, buf_a, sem_a)
            compute(buf_a)
            issue_out(even, buf_a, sem_oa)

            wait_gathers(pp, 1, buf_b, sem_b)
            compute(buf_b)
            issue_out(even + 1, buf_b, sem_ob)

            wait_out(even, buf_a, sem_oa)

            @pl.when(pp < PAIR_PER_STAGE - 1)
            def _():
                issue_gathers(pp + 1, 0, buf_a, sem_a)
            return carry

        lax.fori_loop(0, PAIR_PER_STAGE, pair_body, 0)

    wait_out(2 * N_PAIR - 1, buf_b, sem_ob)


def kernel(value, depth, position, W_value, W_depth, W_pos):
    nv = W_value.shape[0]
    nd = W_depth.shape[0]
    npos = W_pos.shape[1]
    w_vd = (W_value[:, None, :] + W_depth[None, :, :]).reshape(-1, EMBED)
    table = jnp.concatenate([w_vd, W_pos[0], W_pos[1], W_pos[2]], axis=0)
    off_p = nv * nd
    idx = jnp.stack(
        [
            value.reshape(-1) * nd + depth.reshape(-1),
            position[..., 0].reshape(-1) + off_p,
            position[..., 1].reshape(-1) + (off_p + npos),
            position[..., 2].reshape(-1) + (off_p + 2 * npos),
        ],
        axis=1,
    ).reshape(NW, N_PAIR, PAIR, FANIN).transpose(0, 1, 3, 2)
    idx = idx.reshape(NW, N_STAGE, IDX_STAGE, EMBED).astype(jnp.int32)
    out = _emb_kernel(table, idx)
    return out.reshape(value.shape[0], value.shape[1], EMBED)
